# Initial kernel scaffold; baseline (speedup 1.0000x reference)
import jax
import jax.numpy as jnp
from jax.experimental import pallas as pl


def kernel(x, wq, wk, wv, wo, wiq, wik, w_ih):
    raise NotImplementedError("write your pallas kernel here")


# speedup vs baseline: 2.9831x; 2.9831x over previous
import jax
import jax.numpy as jnp
from jax.experimental import pallas as pl

S = 2048
D = 2048
H, DH = 16, 128
HI, DI = 4, 64
TOPK = 512
NEG = -1e30

QB = 128
NB = 512
WCOLS = 3 * H * DH + HI * DI + DI
WPAD = 6656


def _bf(a):
    return a.astype(jnp.bfloat16)


def _proj_kernel(x_ref, w_ref, o_ref):
    o_ref[...] = jnp.dot(_bf(x_ref[...]), _bf(w_ref[...]),
                         preferred_element_type=jnp.float32)


def _attn_kernel(q_ref, k_ref, v_ref, iq_ref, ik_ref, wih_ref, o_ref):
    i = pl.program_id(0)

    ik = ik_ref[:, :DI]
    isc = None
    for h in range(HI):
        iq_h = iq_ref[:, h * DI:(h + 1) * DI]
        s = jax.lax.dot_general(_bf(iq_h), _bf(ik), (((1,), (1,)), ((), ())),
                                preferred_element_type=jnp.float32)
        r = (_bf(jnp.maximum(s, 0.0)).astype(jnp.float32)
             * _bf(wih_ref[0, h]).astype(jnp.float32))
        isc = r if isc is None else isc + r

    rows = i * QB + jax.lax.broadcasted_iota(jnp.int32, (QB, S), 0)
    cols = jax.lax.broadcasted_iota(jnp.int32, (QB, S), 1)
    causal = cols <= rows
    isc = jnp.where(causal, isc, NEG)

    bits = jax.lax.bitcast_convert_type(isc, jnp.int32)
    key = jnp.where(bits < 0, bits ^ jnp.int32(0x7FFFFFFF), bits)
    cand = jnp.full((QB, 1), jnp.int32(-2**31))
    for b in range(31, -1, -1):
        mask_b = jnp.int32(-2**31) if b == 31 else jnp.int32(1 << b)
        trial = cand ^ mask_b
        cnt = jnp.sum((key >= trial).astype(jnp.int32), axis=1,
                      keepdims=True)
        cand = jnp.where(cnt >= TOPK, trial, cand)
    sel = (key >= cand) & causal
    madd = jnp.where(sel, 0.0, NEG).astype(jnp.float32)

    scale = 1.0 / jnp.sqrt(jnp.float32(DH))
    for h in range(H):
        q_h = q_ref[:, h * DH:(h + 1) * DH]
        k_h = k_ref[:, h * DH:(h + 1) * DH]
        v_h = v_ref[:, h * DH:(h + 1) * DH]
        logits = jax.lax.dot_general(_bf(q_h), _bf(k_h),
                                     (((1,), (1,)), ((), ())),
                                     preferred_element_type=jnp.float32)
        logits = logits * scale + madd
        m = jnp.max(logits, axis=1, keepdims=True)
        p = jnp.exp(logits - m)
        denom = jnp.sum(p, axis=1, keepdims=True)
        o_h = jnp.dot(_bf(p), _bf(v_h),
                      preferred_element_type=jnp.float32) / denom
        o_ref[:, h * DH:(h + 1) * DH] = o_h


def kernel(x, wq, wk, wv, wo, wiq, wik, w_ih):
    x2 = x[0]
    W = jnp.concatenate([wq, wk, wv, wiq, wik], axis=1)
    W = jnp.pad(W, ((0, 0), (0, WPAD - WCOLS)))

    qkv = pl.pallas_call(
        _proj_kernel,
        grid=(WPAD // NB,),
        in_specs=[pl.BlockSpec((S, D), lambda j: (0, 0)),
                  pl.BlockSpec((D, NB), lambda j: (0, j))],
        out_specs=pl.BlockSpec((S, NB), lambda j: (0, j)),
        out_shape=jax.ShapeDtypeStruct((S, WPAD), jnp.float32),
    )(x2, W)

    wih2 = jnp.pad(w_ih.reshape(1, HI), ((0, 0), (0, 128 - HI)))
    att = pl.pallas_call(
        _attn_kernel,
        grid=(S // QB,),
        in_specs=[
            pl.BlockSpec((QB, H * DH), lambda i: (i, 0)),
            pl.BlockSpec((S, H * DH), lambda i: (0, 1)),
            pl.BlockSpec((S, H * DH), lambda i: (0, 2)),
            pl.BlockSpec((QB, HI * DI), lambda i: (i, 24)),
            pl.BlockSpec((S, 128), lambda i: (0, 50)),
            pl.BlockSpec((1, 128), lambda i: (0, 0)),
        ],
        out_specs=pl.BlockSpec((QB, H * DH), lambda i: (i, 0)),
        out_shape=jax.ShapeDtypeStruct((S, H * DH), jnp.float32),
    )(qkv, qkv, qkv, qkv, qkv, wih2)

    out = pl.pallas_call(
        _proj_kernel,
        grid=(D // NB,),
        in_specs=[pl.BlockSpec((S, H * DH), lambda j: (0, 0)),
                  pl.BlockSpec((H * DH, NB), lambda j: (0, j))],
        out_specs=pl.BlockSpec((S, NB), lambda j: (0, j)),
        out_shape=jax.ShapeDtypeStruct((S, D), jnp.float32),
    )(att, wo)
    return out.reshape(1, S, D)


# speedup vs baseline: 3.1801x; 1.0660x over previous
import jax
import jax.numpy as jnp
from jax.experimental import pallas as pl

S = 2048
D = 2048
H, DH = 16, 128
HI, DI = 4, 64
TOPK = 512
NEG = -1e30

QB = 128
NB = 512
WCOLS = 3 * H * DH + HI * DI + DI
WPAD = 6656


def _bf(a):
    return a.astype(jnp.bfloat16)


def _proj_kernel(x_ref, w_ref, o_ref):
    o_ref[...] = _bf(jnp.dot(x_ref[...], w_ref[...],
                             preferred_element_type=jnp.float32))


def _attn_kernel(q_ref, k_ref, v_ref, iq_ref, ik_ref, wih_ref, wo_ref, o_ref):
    i = pl.program_id(0)

    ik = ik_ref[:, :DI]
    isc = None
    for h in range(HI):
        iq_h = iq_ref[:, h * DI:(h + 1) * DI]
        s = jax.lax.dot_general(iq_h, ik, (((1,), (1,)), ((), ())),
                                preferred_element_type=jnp.float32)
        r = (_bf(jnp.maximum(s, 0.0)).astype(jnp.float32)
             * _bf(wih_ref[0, h]).astype(jnp.float32))
        isc = r if isc is None else isc + r

    rows = i * QB + jax.lax.broadcasted_iota(jnp.int32, (QB, S), 0)
    cols = jax.lax.broadcasted_iota(jnp.int32, (QB, S), 1)
    causal = cols <= rows
    isc = jnp.where(causal, isc, NEG)

    bits = jax.lax.bitcast_convert_type(isc, jnp.int32)
    key = jnp.where(bits < 0, bits ^ jnp.int32(0x7FFFFFFF), bits)
    cand = jnp.full((QB, 1), jnp.int32(-2**31))
    for b in range(31, -1, -1):
        mask_b = jnp.int32(-2**31) if b == 31 else jnp.int32(1 << b)
        trial = cand ^ mask_b
        cnt = jnp.sum((key >= trial).astype(jnp.int32), axis=1,
                      keepdims=True)
        cand = jnp.where(cnt >= TOPK, trial, cand)
    sel = (key >= cand) & causal
    madd = jnp.where(sel, 0.0, NEG).astype(jnp.float32)

    scale = 1.0 / jnp.sqrt(jnp.float32(DH))
    outs = []
    for h in range(H):
        q_h = q_ref[:, h * DH:(h + 1) * DH]
        k_h = k_ref[:, h * DH:(h + 1) * DH]
        v_h = v_ref[:, h * DH:(h + 1) * DH]
        logits = jax.lax.dot_general(q_h, k_h, (((1,), (1,)), ((), ())),
                                     preferred_element_type=jnp.float32)
        logits = logits * scale + madd
        m = jnp.max(logits, axis=1, keepdims=True)
        p = jnp.exp(logits - m)
        denom = jnp.sum(p, axis=1, keepdims=True)
        o_h = jnp.dot(_bf(p), v_h,
                      preferred_element_type=jnp.float32) / denom
        outs.append(_bf(o_h))
    ob = jnp.concatenate(outs, axis=1)
    o_ref[...] = jnp.dot(ob, wo_ref[...], preferred_element_type=jnp.float32)


def kernel(x, wq, wk, wv, wo, wiq, wik, w_ih):
    xb = _bf(x[0])
    W = jnp.concatenate([_bf(wq), _bf(wk), _bf(wv), _bf(wiq), _bf(wik)],
                        axis=1)
    W = jnp.pad(W, ((0, 0), (0, WPAD - WCOLS)))

    qkv = pl.pallas_call(
        _proj_kernel,
        grid=(WPAD // NB,),
        in_specs=[pl.BlockSpec((S, D), lambda j: (0, 0)),
                  pl.BlockSpec((D, NB), lambda j: (0, j))],
        out_specs=pl.BlockSpec((S, NB), lambda j: (0, j)),
        out_shape=jax.ShapeDtypeStruct((S, WPAD), jnp.bfloat16),
    )(xb, W)

    wih2 = jnp.pad(w_ih.reshape(1, HI), ((0, 0), (0, 128 - HI)))
    out = pl.pallas_call(
        _attn_kernel,
        grid=(S // QB,),
        in_specs=[
            pl.BlockSpec((QB, H * DH), lambda i: (i, 0)),
            pl.BlockSpec((S, H * DH), lambda i: (0, 1)),
            pl.BlockSpec((S, H * DH), lambda i: (0, 2)),
            pl.BlockSpec((QB, HI * DI), lambda i: (i, 24)),
            pl.BlockSpec((S, 128), lambda i: (0, 50)),
            pl.BlockSpec((1, 128), lambda i: (0, 0)),
            pl.BlockSpec((D, D), lambda i: (0, 0)),
        ],
        out_specs=pl.BlockSpec((QB, D), lambda i: (i, 0)),
        out_shape=jax.ShapeDtypeStruct((S, D), jnp.float32),
    )(qkv, qkv, qkv, qkv, qkv, wih2, _bf(wo))
    return out.reshape(1, S, D)


# speedup vs baseline: 3.4945x; 1.0989x over previous
import jax
import jax.numpy as jnp
from jax.experimental import pallas as pl

S = 2048
D = 2048
H, DH = 16, 128
HI, DI = 4, 64
TOPK = 512
NEG = -1e30

QB = 256
NB = 512
WCOLS = 3 * H * DH + HI * DI + DI
WPAD = 6656


def _bf(a):
    return a.astype(jnp.bfloat16)


def _proj_kernel(x_ref, w_ref, o_ref):
    o_ref[...] = _bf(jnp.dot(x_ref[...], w_ref[...],
                             preferred_element_type=jnp.float32))


def _attn_kernel(q_ref, k_ref, v_ref, iq_ref, ik_ref, wih_ref, wo_ref, o_ref):
    i = pl.program_id(0)

    ik = ik_ref[:, :DI]
    isc = None
    for h in range(HI):
        iq_h = iq_ref[:, h * DI:(h + 1) * DI]
        s = jax.lax.dot_general(iq_h, ik, (((1,), (1,)), ((), ())),
                                preferred_element_type=jnp.float32)
        r = (_bf(jnp.maximum(s, 0.0)).astype(jnp.float32)
             * _bf(wih_ref[0, h]).astype(jnp.float32))
        isc = r if isc is None else isc + r

    rows = i * QB + jax.lax.broadcasted_iota(jnp.int32, (QB, S), 0)
    cols = jax.lax.broadcasted_iota(jnp.int32, (QB, S), 1)
    causal = cols <= rows
    isc = jnp.where(causal, isc, NEG)

    bits = jax.lax.bitcast_convert_type(isc, jnp.int32)
    key = jnp.where(bits < 0, bits ^ jnp.int32(0x7FFFFFFF), bits)
    cand = jnp.full((QB, 1), jnp.int32(-2**31))
    for b in range(31, -1, -1):
        mask_b = jnp.int32(-2**31) if b == 31 else jnp.int32(1 << b)
        trial = cand ^ mask_b
        cnt = jnp.sum((key >= trial).astype(jnp.int32), axis=1,
                      keepdims=True)
        cand = jnp.where(cnt >= TOPK, trial, cand)
    sel = (key >= cand) & causal
    madd = jnp.where(sel, 0.0, NEG).astype(jnp.float32)

    scale = 1.0 / jnp.sqrt(jnp.float32(DH))
    outs = []
    for h in range(H):
        q_h = q_ref[:, h * DH:(h + 1) * DH]
        k_h = k_ref[:, h * DH:(h + 1) * DH]
        v_h = v_ref[:, h * DH:(h + 1) * DH]
        logits = jax.lax.dot_general(q_h, k_h, (((1,), (1,)), ((), ())),
                                     preferred_element_type=jnp.float32)
        logits = logits * scale + madd
        m = jnp.max(logits, axis=1, keepdims=True)
        p = jnp.exp(logits - m)
        denom = jnp.sum(p, axis=1, keepdims=True)
        o_h = jnp.dot(_bf(p), v_h,
                      preferred_element_type=jnp.float32) / denom
        outs.append(_bf(o_h))
    ob = jnp.concatenate(outs, axis=1)
    o_ref[...] = jnp.dot(ob, wo_ref[...], preferred_element_type=jnp.float32)


def kernel(x, wq, wk, wv, wo, wiq, wik, w_ih):
    xb = _bf(x[0])
    W = jnp.concatenate([_bf(wq), _bf(wk), _bf(wv), _bf(wiq), _bf(wik)],
                        axis=1)
    W = jnp.pad(W, ((0, 0), (0, WPAD - WCOLS)))

    qkv = pl.pallas_call(
        _proj_kernel,
        grid=(WPAD // NB,),
        in_specs=[pl.BlockSpec((S, D), lambda j: (0, 0)),
                  pl.BlockSpec((D, NB), lambda j: (0, j))],
        out_specs=pl.BlockSpec((S, NB), lambda j: (0, j)),
        out_shape=jax.ShapeDtypeStruct((S, WPAD), jnp.bfloat16),
    )(xb, W)

    wih2 = jnp.pad(w_ih.reshape(1, HI), ((0, 0), (0, 128 - HI)))
    out = pl.pallas_call(
        _attn_kernel,
        grid=(S // QB,),
        in_specs=[
            pl.BlockSpec((QB, H * DH), lambda i: (i, 0)),
            pl.BlockSpec((S, H * DH), lambda i: (0, 1)),
            pl.BlockSpec((S, H * DH), lambda i: (0, 2)),
            pl.BlockSpec((QB, HI * DI), lambda i: (i, 24)),
            pl.BlockSpec((S, 128), lambda i: (0, 50)),
            pl.BlockSpec((1, 128), lambda i: (0, 0)),
            pl.BlockSpec((D, D), lambda i: (0, 0)),
        ],
        out_specs=pl.BlockSpec((QB, D), lambda i: (i, 0)),
        out_shape=jax.ShapeDtypeStruct((S, D), jnp.float32),
    )(qkv, qkv, qkv, qkv, qkv, wih2, _bf(wo))
    return out.reshape(1, S, D)
